# Initial kernel scaffold; baseline (speedup 1.0000x reference)
import jax
import jax.numpy as jnp
from jax.experimental import pallas as pl


def kernel(logits):
    raise NotImplementedError("write your pallas kernel here")


# speedup vs baseline: 149.9340x; 149.9340x over previous
import jax
import jax.numpy as jnp
from jax.experimental import pallas as pl

_ROWS = 8
_ONE_BITS = 0x3F800000
_TOP_K = 64
_TOP_P = 0.95


def _body(x_ref, conf_ref, idx_ref):
    x = x_ref[...]
    m = jnp.max(x, axis=1, keepdims=True)

    col = jax.lax.broadcasted_iota(jnp.int32, x.shape, 1)
    big = jnp.int32(0x7FFFFFFF)
    idx = jnp.min(jnp.where(x == m, col, big), axis=1, keepdims=True)

    e = jnp.exp(x - m)
    z = jnp.sum(e, axis=1, keepdims=True)
    target = jnp.float32(_TOP_P) * z

    def step(_, lo_hi):
        lo, hi = lo_hi
        mid = (lo + hi + 1) >> 1
        u = jax.lax.bitcast_convert_type(mid, jnp.float32)
        ge = e >= u
        cnt = jnp.sum(ge.astype(jnp.int32), axis=1, keepdims=True)
        s = jnp.sum(jnp.where(ge, e, 0.0), axis=1, keepdims=True)
        p = (cnt >= _TOP_K) | (s > target)
        lo = jnp.where(p, mid, lo)
        hi = jnp.where(p, hi, mid - 1)
        return lo, hi

    lo0 = jnp.zeros(m.shape, jnp.int32)
    hi0 = jnp.full(m.shape, _ONE_BITS, jnp.int32)
    lo, _ = jax.lax.fori_loop(0, 31, step, (lo0, hi0))

    u = jax.lax.bitcast_convert_type(lo, jnp.float32)
    d = jnp.sum(jnp.where(e >= u, e, 0.0), axis=1, keepdims=True)
    conf_ref[...] = 1.0 / d
    idx_ref[...] = idx


def kernel(logits):
    n, v = logits.shape
    grid = n // _ROWS
    conf, idx = pl.pallas_call(
        _body,
        grid=(grid,),
        in_specs=[pl.BlockSpec((_ROWS, v), lambda i: (i, 0))],
        out_specs=[
            pl.BlockSpec((_ROWS, 1), lambda i: (i, 0)),
            pl.BlockSpec((_ROWS, 1), lambda i: (i, 0)),
        ],
        out_shape=[
            jax.ShapeDtypeStruct((n, 1), jnp.float32),
            jax.ShapeDtypeStruct((n, 1), jnp.int32),
        ],
    )(logits)
    return conf.reshape(n), idx.reshape(n)
